# baseline (speedup 1.0000x reference)
import jax
import jax.numpy as jnp
from jax.experimental import pallas as pl
from jax.experimental.pallas import tpu as pltpu

B, C, H, W = 8, 320, 48, 48
HW = H * W
LANES = 128
RPC = HW // LANES
CB = 64
RB = CB * RPC


def _scale_body(n_ref, g_ref, x_ref, o_ref):
    o_ref[...] = jnp.abs(g_ref[0]) * x_ref[...]


def kernel(x, n, gain_matrix):
    x2 = x.reshape(B * C * RPC, LANES)
    g_rep = jnp.repeat(gain_matrix, RPC, axis=1).reshape(B, C * RPC, 1)
    nblk = C // CB
    out = pl.pallas_call(
        _scale_body,
        grid_spec=pltpu.PrefetchScalarGridSpec(
            num_scalar_prefetch=1,
            grid=(B, nblk),
            in_specs=[
                pl.BlockSpec((1, RB, 1), lambda b, c, n_ref: (n_ref[b], c, 0)),
                pl.BlockSpec((RB, LANES), lambda b, c, n_ref: (b * nblk + c, 0)),
            ],
            out_specs=pl.BlockSpec((RB, LANES), lambda b, c, n_ref: (b * nblk + c, 0)),
        ),
        out_shape=jax.ShapeDtypeStruct((B * C * RPC, LANES), jnp.float32),
    )(n.astype(jnp.int32), g_rep, x2)
    return out.reshape(B, C, H, W)


# speedup vs baseline: 4.4859x; 4.4859x over previous
import jax
import jax.numpy as jnp
from jax.experimental import pallas as pl
from jax.experimental.pallas import tpu as pltpu

B, C, H, W = 8, 320, 48, 48
HW = H * W
CB = 64


def _scale_body(n_ref, g_ref, x_ref, o_ref):
    g = jnp.abs(g_ref[0, 0, 0])
    o_ref[...] = g[None, :, None] * x_ref[...]


def kernel(x, n, gain_matrix):
    x3 = x.reshape(B, C, HW)
    g4 = gain_matrix.reshape(B, C // CB, 1, CB)
    out = pl.pallas_call(
        _scale_body,
        grid_spec=pltpu.PrefetchScalarGridSpec(
            num_scalar_prefetch=1,
            grid=(B, C // CB),
            in_specs=[
                pl.BlockSpec((1, 1, 1, CB), lambda b, c, n_ref: (n_ref[b], c, 0, 0)),
                pl.BlockSpec((1, CB, HW), lambda b, c, n_ref: (b, c, 0)),
            ],
            out_specs=pl.BlockSpec((1, CB, HW), lambda b, c, n_ref: (b, c, 0)),
        ),
        out_shape=jax.ShapeDtypeStruct((B, C, HW), jnp.float32),
    )(n.astype(jnp.int32), g4, x3)
    return out.reshape(B, C, H, W)
